# Initial kernel scaffold; baseline (speedup 1.0000x reference)
import jax
import jax.numpy as jnp
from jax.experimental import pallas as pl


def kernel(s, v, radial_embeddings, f_cut, unit_vectors, edge_index, W1, b1, W2, b2, Wr, br):
    raise NotImplementedError("write your pallas kernel here")


# speedup vs baseline: 1.6956x; 1.6956x over previous
import functools

import jax
import jax.numpy as jnp
from jax import lax
from jax.experimental import pallas as pl
from jax.experimental.pallas import tpu as pltpu
from jax.experimental.pallas import tpu_sc as plsc

N = 10000
E = 320000
F = 128
NC = 2
NS = 16
NW = NC * NS
EPW = E // NW
C = 40
NCHUNK = EPW // C
ZCH = C
NZCH = N // ZCH
ZPT = (NZCH + NS - 1) // NS
KG = F // 16


def _phi_kernel(s_ref, w1_ref, b1_ref, w2_ref, b2_ref, os_ref, ovv_ref, ovs_ref):
    h = jnp.dot(s_ref[...], w1_ref[...], preferred_element_type=jnp.float32) + b1_ref[...]
    h = h * jax.nn.sigmoid(h)
    ph = jnp.dot(h, w2_ref[...], preferred_element_type=jnp.float32) + b2_ref[...]
    os_ref[...] = ph[:, :F]
    ovv_ref[...] = ph[:, F:2 * F]
    ovs_ref[...] = ph[:, 2 * F:]


def _wm_kernel(rad_ref, fc_ref, fu_ref, wr_ref, br_ref, ws_ref, wvv_ref, wvs_ref):
    w = jnp.dot(rad_ref[...], wr_ref[...], preferred_element_type=jnp.float32) + br_ref[...]
    fc = fc_ref[...]
    fu = fu_ref[...]
    ws_ref[...] = w[:, :F] * fc
    wvv_ref[...] = w[:, F:2 * F] * fc
    wvs_ref[...] = w[:, 2 * F:] * (fc * fu)


def _comb_kernel(s_ref, a0_ref, a1_ref, v_ref, c0_ref, c1_ref, os_ref, ov_ref):
    os_ref[...] = s_ref[...] + a0_ref[...] + a1_ref[...]
    ov_ref[...] = v_ref[...] + c0_ref[...] + c1_ref[...]


def _sc_body(phis, phivv, phivs, vtab, ws, wvv, wvs, idxi_hbm, idxj_hbm,
             outs, outv,
             idxi_v, idxj_v, b1, b2, b3, b4, b5, b6, acc, sem):
    ci = lax.axis_index("c")
    si = lax.axis_index("s")
    wid = si * NC + ci
    wbase = wid * EPW

    zeros16 = jnp.zeros((16,), jnp.float32)

    def zero_acc():
        def zfill(j, _):
            for k in range(KG):
                b6[j, pl.ds(k * 16, 16)] = zeros16
            return 0

        lax.fori_loop(0, ZCH, zfill, 0)
        for t in range(ZPT):
            q = si * ZPT + t
            @pl.when(q < NZCH)
            def _():
                pltpu.sync_copy(b6, acc.at[pl.ds(q * ZCH, ZCH)])

    def drain(dst):
        for t in range(ZPT):
            q = si * ZPT + t
            @pl.when(q < NZCH)
            def _():
                pltpu.sync_copy(acc.at[pl.ds(q * ZCH, ZCH)], b6)
                pltpu.sync_copy(b6, dst.at[ci, pl.ds(q * ZCH, ZCH)])

    zero_acc()
    plsc.subcore_barrier()

    def chunk_a(kc, _):
        base = wbase + kc * C
        pltpu.sync_copy(idxj_hbm.at[pl.ds(base, C)], idxj_v)
        pltpu.sync_copy(idxi_hbm.at[pl.ds(base, C)], idxi_v)
        g = pltpu.async_copy(phis.at[idxj_v], b1, sem)
        pltpu.sync_copy(ws.at[pl.ds(base, C)], b4)
        g.wait()

        def mulrow(j, _):
            for k in range(KG):
                sl = pl.ds(k * 16, 16)
                b6[j, sl] = b1[j, sl] * b4[j, sl]
            return 0

        lax.fori_loop(0, C, mulrow, 0)
        pltpu.sync_copy(b6, acc.at[idxi_v], add=True)
        return 0

    lax.fori_loop(0, NCHUNK, chunk_a, 0)
    plsc.subcore_barrier()
    drain(outs)
    plsc.subcore_barrier()

    zero_acc()
    plsc.subcore_barrier()

    def chunk_b(kc, _):
        base = wbase + kc * C
        pltpu.sync_copy(idxj_hbm.at[pl.ds(base, C)], idxj_v)
        pltpu.sync_copy(idxi_hbm.at[pl.ds(base, C)], idxi_v)
        g1 = pltpu.async_copy(phivv.at[idxj_v], b1, sem)
        g2 = pltpu.async_copy(phivs.at[idxj_v], b2, sem)
        g3 = pltpu.async_copy(vtab.at[idxj_v], b3, sem)
        pltpu.sync_copy(wvv.at[pl.ds(base, C)], b4)
        pltpu.sync_copy(wvs.at[pl.ds(base, C)], b5)
        g1.wait()
        g2.wait()
        g3.wait()

        def mulrow(j, _):
            for k in range(KG):
                sl = pl.ds(k * 16, 16)
                b6[j, sl] = b3[j, sl] * (b1[j, sl] * b4[j, sl]) + b2[j, sl] * b5[j, sl]
            return 0

        lax.fori_loop(0, C, mulrow, 0)
        pltpu.sync_copy(b6, acc.at[idxi_v], add=True)
        return 0

    lax.fori_loop(0, NCHUNK, chunk_b, 0)
    plsc.subcore_barrier()
    drain(outv)


def kernel(s, v, radial_embeddings, f_cut, unit_vectors, edge_index, W1, b1, W2, b2, Wr, br):
    f32 = jnp.float32

    BN = 1000
    phi_s, phi_vv, phi_vs = pl.pallas_call(
        _phi_kernel,
        grid=(N // BN,),
        in_specs=[
            pl.BlockSpec((BN, F), lambda i: (i, 0)),
            pl.BlockSpec((F, F), lambda i: (0, 0)),
            pl.BlockSpec((1, F), lambda i: (0, 0)),
            pl.BlockSpec((F, 3 * F), lambda i: (0, 0)),
            pl.BlockSpec((1, 3 * F), lambda i: (0, 0)),
        ],
        out_specs=[pl.BlockSpec((BN, F), lambda i: (i, 0))] * 3,
        out_shape=[jax.ShapeDtypeStruct((N, F), f32)] * 3,
    )(s, W1, b1.reshape(1, F), W2, b2.reshape(1, 3 * F))

    BE = 4000
    R = radial_embeddings.shape[1]
    w_s, w_vv, w_vs = pl.pallas_call(
        _wm_kernel,
        grid=(E // BE,),
        in_specs=[
            pl.BlockSpec((BE, R), lambda i: (i, 0)),
            pl.BlockSpec((BE, 1), lambda i: (i, 0)),
            pl.BlockSpec((BE, 1), lambda i: (i, 0)),
            pl.BlockSpec((R, 3 * F), lambda i: (0, 0)),
            pl.BlockSpec((1, 3 * F), lambda i: (0, 0)),
        ],
        out_specs=[pl.BlockSpec((BE, F), lambda i: (i, 0))] * 3,
        out_shape=[jax.ShapeDtypeStruct((E, F), f32)] * 3,
    )(radial_embeddings, f_cut.reshape(E, 1), unit_vectors.reshape(E, 1),
      Wr, br.reshape(1, 3 * F))

    idx_i = edge_index[0]
    idx_j = edge_index[1]
    mesh = plsc.VectorSubcoreMesh(core_axis_name="c", subcore_axis_name="s")
    ps, pv = pl.kernel(
        _sc_body,
        out_type=(jax.ShapeDtypeStruct((NC, N, F), f32),
                  jax.ShapeDtypeStruct((NC, N, F), f32)),
        mesh=mesh,
        scratch_types=[
            pltpu.VMEM((C,), jnp.int32),
            pltpu.VMEM((C,), jnp.int32),
            pltpu.VMEM((C, F), f32),
            pltpu.VMEM((C, F), f32),
            pltpu.VMEM((C, F), f32),
            pltpu.VMEM((C, F), f32),
            pltpu.VMEM((C, F), f32),
            pltpu.VMEM((C, F), f32),
            pltpu.VMEM_SHARED((N, F), f32),
            pltpu.SemaphoreType.DMA,
        ],
    )(phi_s, phi_vv, phi_vs, v, w_s, w_vv, w_vs, idx_i, idx_j)

    out_s, out_v = pl.pallas_call(
        _comb_kernel,
        grid=(N // BN,),
        in_specs=[pl.BlockSpec((BN, F), lambda i: (i, 0))] * 6,
        out_specs=[pl.BlockSpec((BN, F), lambda i: (i, 0))] * 2,
        out_shape=[jax.ShapeDtypeStruct((N, F), f32)] * 2,
    )(s, ps[0], ps[1], v, pv[0], pv[1])
    return (out_s, out_v)


# speedup vs baseline: 1.7179x; 1.0132x over previous
import functools

import jax
import jax.numpy as jnp
from jax import lax
from jax.experimental import pallas as pl
from jax.experimental.pallas import tpu as pltpu
from jax.experimental.pallas import tpu_sc as plsc

N = 10000
E = 320000
F = 128
NC = 2
NS = 16
NW = NC * NS
EPW = E // NW
C = 40
NCHUNK = EPW // C
ZCH = C
NZCH = N // ZCH
ZPT = (NZCH + NS - 1) // NS
KG = F // 16


def _phi_kernel(s_ref, w1_ref, b1_ref, w2_ref, b2_ref, os_ref, ovv_ref, ovs_ref):
    h = jnp.dot(s_ref[...], w1_ref[...], preferred_element_type=jnp.float32) + b1_ref[...]
    h = h * jax.nn.sigmoid(h)
    ph = jnp.dot(h, w2_ref[...], preferred_element_type=jnp.float32) + b2_ref[...]
    os_ref[...] = ph[:, :F]
    ovv_ref[...] = ph[:, F:2 * F]
    ovs_ref[...] = ph[:, 2 * F:]


def _wm_kernel(rad_ref, wr_ref, br_ref, ws_ref, wvv_ref, wvs_ref):
    w = jnp.dot(rad_ref[...], wr_ref[...], preferred_element_type=jnp.float32) + br_ref[...]
    ws_ref[...] = w[:, :F]
    wvv_ref[...] = w[:, F:2 * F]
    wvs_ref[...] = w[:, 2 * F:]


def _comb_kernel(s_ref, a0_ref, a1_ref, v_ref, c0_ref, c1_ref, os_ref, ov_ref):
    os_ref[...] = s_ref[...] + a0_ref[...] + a1_ref[...]
    ov_ref[...] = v_ref[...] + c0_ref[...] + c1_ref[...]


def _sc_body(phis, phivv, phivs, vtab, ws, wvv, wvs, idxi_hbm, idxj_hbm,
             fc_hbm, fu_hbm,
             outs, outv,
             idxi_v, idxj_v, fc_v, fu_v, b1, b2, b3, b4, b5, b6, acc, sem):
    ci = lax.axis_index("c")
    si = lax.axis_index("s")
    wid = si * NC + ci
    wbase = wid * EPW

    zeros16 = jnp.zeros((16,), jnp.float32)

    def zero_acc():
        def zfill(j, _):
            for k in range(KG):
                b6[j, pl.ds(k * 16, 16)] = zeros16
            return 0

        lax.fori_loop(0, ZCH, zfill, 0)
        for t in range(ZPT):
            q = si * ZPT + t
            @pl.when(q < NZCH)
            def _():
                pltpu.sync_copy(b6, acc.at[pl.ds(q * ZCH, ZCH)])

    def drain(dst):
        for t in range(ZPT):
            q = si * ZPT + t
            @pl.when(q < NZCH)
            def _():
                pltpu.sync_copy(acc.at[pl.ds(q * ZCH, ZCH)], b6)
                pltpu.sync_copy(b6, dst.at[ci, pl.ds(q * ZCH, ZCH)])

    zero_acc()
    plsc.subcore_barrier()

    def chunk_a(kc, _):
        base = wbase + kc * C
        pltpu.sync_copy(idxj_hbm.at[pl.ds(base, C)], idxj_v)
        pltpu.sync_copy(idxi_hbm.at[pl.ds(base, C)], idxi_v)
        g = pltpu.async_copy(phis.at[idxj_v], b1, sem)
        pltpu.sync_copy(ws.at[pl.ds(base, C)], b4)
        pltpu.sync_copy(fc_hbm.at[pl.ds(base, C)], fc_v.at[pl.ds(0, C)])
        g.wait()

        def mulrow(j, _):
            fcb = jnp.full((16,), fc_v[pl.ds(j, 16)][0], jnp.float32)
            for k in range(KG):
                sl = pl.ds(k * 16, 16)
                b6[j, sl] = b1[j, sl] * b4[j, sl] * fcb
            return 0

        lax.fori_loop(0, C, mulrow, 0)
        pltpu.sync_copy(b6, acc.at[idxi_v], add=True)
        return 0

    lax.fori_loop(0, NCHUNK, chunk_a, 0)
    plsc.subcore_barrier()
    drain(outs)
    plsc.subcore_barrier()

    zero_acc()
    plsc.subcore_barrier()

    def chunk_b(kc, _):
        base = wbase + kc * C
        pltpu.sync_copy(idxj_hbm.at[pl.ds(base, C)], idxj_v)
        pltpu.sync_copy(idxi_hbm.at[pl.ds(base, C)], idxi_v)
        g1 = pltpu.async_copy(phivv.at[idxj_v], b1, sem)
        g2 = pltpu.async_copy(phivs.at[idxj_v], b2, sem)
        g3 = pltpu.async_copy(vtab.at[idxj_v], b3, sem)
        pltpu.sync_copy(wvv.at[pl.ds(base, C)], b4)
        pltpu.sync_copy(wvs.at[pl.ds(base, C)], b5)
        pltpu.sync_copy(fc_hbm.at[pl.ds(base, C)], fc_v.at[pl.ds(0, C)])
        pltpu.sync_copy(fu_hbm.at[pl.ds(base, C)], fu_v.at[pl.ds(0, C)])
        g1.wait()
        g2.wait()
        g3.wait()

        def mulrow(j, _):
            fcb = jnp.full((16,), fc_v[pl.ds(j, 16)][0], jnp.float32)
            fub = jnp.full((16,), fu_v[pl.ds(j, 16)][0], jnp.float32)
            for k in range(KG):
                sl = pl.ds(k * 16, 16)
                b6[j, sl] = (b3[j, sl] * (b1[j, sl] * b4[j, sl])
                             + b2[j, sl] * b5[j, sl] * fub) * fcb
            return 0

        lax.fori_loop(0, C, mulrow, 0)
        pltpu.sync_copy(b6, acc.at[idxi_v], add=True)
        return 0

    lax.fori_loop(0, NCHUNK, chunk_b, 0)
    plsc.subcore_barrier()
    drain(outv)


def kernel(s, v, radial_embeddings, f_cut, unit_vectors, edge_index, W1, b1, W2, b2, Wr, br):
    f32 = jnp.float32

    BN = 1000
    phi_s, phi_vv, phi_vs = pl.pallas_call(
        _phi_kernel,
        grid=(N // BN,),
        in_specs=[
            pl.BlockSpec((BN, F), lambda i: (i, 0)),
            pl.BlockSpec((F, F), lambda i: (0, 0)),
            pl.BlockSpec((1, F), lambda i: (0, 0)),
            pl.BlockSpec((F, 3 * F), lambda i: (0, 0)),
            pl.BlockSpec((1, 3 * F), lambda i: (0, 0)),
        ],
        out_specs=[pl.BlockSpec((BN, F), lambda i: (i, 0))] * 3,
        out_shape=[jax.ShapeDtypeStruct((N, F), f32)] * 3,
    )(s, W1, b1.reshape(1, F), W2, b2.reshape(1, 3 * F))

    BE = 4000
    R = radial_embeddings.shape[1]
    w_s, w_vv, w_vs = pl.pallas_call(
        _wm_kernel,
        grid=(E // BE,),
        in_specs=[
            pl.BlockSpec((BE, R), lambda i: (i, 0)),
            pl.BlockSpec((R, 3 * F), lambda i: (0, 0)),
            pl.BlockSpec((1, 3 * F), lambda i: (0, 0)),
        ],
        out_specs=[pl.BlockSpec((BE, F), lambda i: (i, 0))] * 3,
        out_shape=[jax.ShapeDtypeStruct((E, F), f32)] * 3,
    )(radial_embeddings, Wr, br.reshape(1, 3 * F))

    idx_i = edge_index[0]
    idx_j = edge_index[1]
    mesh = plsc.VectorSubcoreMesh(core_axis_name="c", subcore_axis_name="s")
    ps, pv = pl.kernel(
        _sc_body,
        out_type=(jax.ShapeDtypeStruct((NC, N, F), f32),
                  jax.ShapeDtypeStruct((NC, N, F), f32)),
        mesh=mesh,
        scratch_types=[
            pltpu.VMEM((C,), jnp.int32),
            pltpu.VMEM((C,), jnp.int32),
            pltpu.VMEM((C + 16,), f32),
            pltpu.VMEM((C + 16,), f32),
            pltpu.VMEM((C, F), f32),
            pltpu.VMEM((C, F), f32),
            pltpu.VMEM((C, F), f32),
            pltpu.VMEM((C, F), f32),
            pltpu.VMEM((C, F), f32),
            pltpu.VMEM((C, F), f32),
            pltpu.VMEM_SHARED((N, F), f32),
            pltpu.SemaphoreType.DMA,
        ],
    )(phi_s, phi_vv, phi_vs, v, w_s, w_vv, w_vs, idx_i, idx_j, f_cut, unit_vectors)

    out_s, out_v = pl.pallas_call(
        _comb_kernel,
        grid=(N // BN,),
        in_specs=[pl.BlockSpec((BN, F), lambda i: (i, 0))] * 6,
        out_specs=[pl.BlockSpec((BN, F), lambda i: (i, 0))] * 2,
        out_shape=[jax.ShapeDtypeStruct((N, F), f32)] * 2,
    )(s, ps[0], ps[1], v, pv[0], pv[1])
    return (out_s, out_v)


# speedup vs baseline: 1.9751x; 1.1497x over previous
import functools

import jax
import jax.numpy as jnp
from jax import lax
from jax.experimental import pallas as pl
from jax.experimental.pallas import tpu as pltpu
from jax.experimental.pallas import tpu_sc as plsc

N = 10000
E = 320000
F = 128
NC = 2
NS = 16
NW = NC * NS
EPW = E // NW
C = 40
NCHUNK = EPW // C
ZCH = C
NZCH = N // ZCH
ZPT = (NZCH + NS - 1) // NS
KG = F // 16


def _phi_kernel(s_ref, w1_ref, b1_ref, w2_ref, b2_ref, os_ref, ovv_ref, ovs_ref):
    h = jnp.dot(s_ref[...], w1_ref[...], preferred_element_type=jnp.float32) + b1_ref[...]
    h = h * jax.nn.sigmoid(h)
    ph = jnp.dot(h, w2_ref[...], preferred_element_type=jnp.float32) + b2_ref[...]
    os_ref[...] = ph[:, :F]
    ovv_ref[...] = ph[:, F:2 * F]
    ovs_ref[...] = ph[:, 2 * F:]


def _wm_kernel(rad_ref, fc_ref, fu_ref, wr_ref, br_ref, ws_ref, wvv_ref, wvs_ref):
    w = jnp.dot(rad_ref[...], wr_ref[...], preferred_element_type=jnp.float32) + br_ref[...]
    fc = jnp.transpose(fc_ref[0], (1, 0))
    fu = jnp.transpose(fu_ref[0], (1, 0))
    ws_ref[...] = w[:, :F] * fc
    wvv_ref[...] = w[:, F:2 * F] * fc
    wvs_ref[...] = w[:, 2 * F:] * (fc * fu)


def _comb_kernel(s_ref, a0_ref, a1_ref, v_ref, c0_ref, c1_ref, os_ref, ov_ref):
    os_ref[...] = s_ref[...] + a0_ref[...] + a1_ref[...]
    ov_ref[...] = v_ref[...] + c0_ref[...] + c1_ref[...]


def _sc_body(phis, phivv, phivs, vtab, ws, wvv, wvs, idxi_hbm, idxj_hbm,
             outs, outv,
             idxi_v, idxj_v, b1, b2, b3, b4, b5, b6, acc, sem):
    ci = lax.axis_index("c")
    si = lax.axis_index("s")
    wid = si * NC + ci
    wbase = wid * EPW

    zeros16 = jnp.zeros((16,), jnp.float32)

    def zero_acc():
        def zfill(j, _):
            for k in range(KG):
                b6[j, pl.ds(k * 16, 16)] = zeros16
            return 0

        lax.fori_loop(0, ZCH, zfill, 0)
        for t in range(ZPT):
            q = si * ZPT + t
            @pl.when(q < NZCH)
            def _():
                pltpu.sync_copy(b6, acc.at[pl.ds(q * ZCH, ZCH)])

    def drain(dst):
        for t in range(ZPT):
            q = si * ZPT + t
            @pl.when(q < NZCH)
            def _():
                pltpu.sync_copy(acc.at[pl.ds(q * ZCH, ZCH)], b6)
                pltpu.sync_copy(b6, dst.at[ci, pl.ds(q * ZCH, ZCH)])

    zero_acc()
    plsc.subcore_barrier()

    def chunk_a(kc, _):
        base = wbase + kc * C
        pltpu.sync_copy(idxj_hbm.at[pl.ds(base, C)], idxj_v)
        pltpu.sync_copy(idxi_hbm.at[pl.ds(base, C)], idxi_v)
        g = pltpu.async_copy(phis.at[idxj_v], b1, sem)
        pltpu.sync_copy(ws.at[pl.ds(base, C)], b4)
        g.wait()

        def mulrow(j, _):
            for k in range(KG):
                sl = pl.ds(k * 16, 16)
                b6[j, sl] = b1[j, sl] * b4[j, sl]
            return 0

        lax.fori_loop(0, C, mulrow, 0)
        pltpu.sync_copy(b6, acc.at[idxi_v], add=True)
        return 0

    lax.fori_loop(0, NCHUNK, chunk_a, 0)
    plsc.subcore_barrier()
    drain(outs)
    plsc.subcore_barrier()

    zero_acc()
    plsc.subcore_barrier()

    def chunk_b(kc, _):
        base = wbase + kc * C
        pltpu.sync_copy(idxj_hbm.at[pl.ds(base, C)], idxj_v)
        pltpu.sync_copy(idxi_hbm.at[pl.ds(base, C)], idxi_v)
        g1 = pltpu.async_copy(phivv.at[idxj_v], b1, sem)
        g2 = pltpu.async_copy(phivs.at[idxj_v], b2, sem)
        g3 = pltpu.async_copy(vtab.at[idxj_v], b3, sem)
        pltpu.sync_copy(wvv.at[pl.ds(base, C)], b4)
        pltpu.sync_copy(wvs.at[pl.ds(base, C)], b5)
        g1.wait()
        g2.wait()
        g3.wait()

        def mulrow(j, _):
            for k in range(KG):
                sl = pl.ds(k * 16, 16)
                b6[j, sl] = b3[j, sl] * (b1[j, sl] * b4[j, sl]) + b2[j, sl] * b5[j, sl]
            return 0

        lax.fori_loop(0, C, mulrow, 0)
        pltpu.sync_copy(b6, acc.at[idxi_v], add=True)
        return 0

    lax.fori_loop(0, NCHUNK, chunk_b, 0)
    plsc.subcore_barrier()
    drain(outv)


def kernel(s, v, radial_embeddings, f_cut, unit_vectors, edge_index, W1, b1, W2, b2, Wr, br):
    f32 = jnp.float32

    BN = 1000
    phi_s, phi_vv, phi_vs = pl.pallas_call(
        _phi_kernel,
        grid=(N // BN,),
        in_specs=[
            pl.BlockSpec((BN, F), lambda i: (i, 0)),
            pl.BlockSpec((F, F), lambda i: (0, 0)),
            pl.BlockSpec((1, F), lambda i: (0, 0)),
            pl.BlockSpec((F, 3 * F), lambda i: (0, 0)),
            pl.BlockSpec((1, 3 * F), lambda i: (0, 0)),
        ],
        out_specs=[pl.BlockSpec((BN, F), lambda i: (i, 0))] * 3,
        out_shape=[jax.ShapeDtypeStruct((N, F), f32)] * 3,
    )(s, W1, b1.reshape(1, F), W2, b2.reshape(1, 3 * F))

    BE = 4000
    R = radial_embeddings.shape[1]
    w_s, w_vv, w_vs = pl.pallas_call(
        _wm_kernel,
        grid=(E // BE,),
        in_specs=[
            pl.BlockSpec((BE, R), lambda i: (i, 0)),
            pl.BlockSpec((1, 1, BE), lambda i: (i, 0, 0)),
            pl.BlockSpec((1, 1, BE), lambda i: (i, 0, 0)),
            pl.BlockSpec((R, 3 * F), lambda i: (0, 0)),
            pl.BlockSpec((1, 3 * F), lambda i: (0, 0)),
        ],
        out_specs=[pl.BlockSpec((BE, F), lambda i: (i, 0))] * 3,
        out_shape=[jax.ShapeDtypeStruct((E, F), f32)] * 3,
    )(radial_embeddings, f_cut.reshape(E // BE, 1, BE), unit_vectors.reshape(E // BE, 1, BE),
      Wr, br.reshape(1, 3 * F))

    idx_i = edge_index[0]
    idx_j = edge_index[1]
    mesh = plsc.VectorSubcoreMesh(core_axis_name="c", subcore_axis_name="s")
    ps, pv = pl.kernel(
        _sc_body,
        out_type=(jax.ShapeDtypeStruct((NC, N, F), f32),
                  jax.ShapeDtypeStruct((NC, N, F), f32)),
        mesh=mesh,
        scratch_types=[
            pltpu.VMEM((C,), jnp.int32),
            pltpu.VMEM((C,), jnp.int32),
            pltpu.VMEM((C, F), f32),
            pltpu.VMEM((C, F), f32),
            pltpu.VMEM((C, F), f32),
            pltpu.VMEM((C, F), f32),
            pltpu.VMEM((C, F), f32),
            pltpu.VMEM((C, F), f32),
            pltpu.VMEM_SHARED((N, F), f32),
            pltpu.SemaphoreType.DMA,
        ],
    )(phi_s, phi_vv, phi_vs, v, w_s, w_vv, w_vs, idx_i, idx_j)

    out_s, out_v = pl.pallas_call(
        _comb_kernel,
        grid=(N // BN,),
        in_specs=[pl.BlockSpec((BN, F), lambda i: (i, 0))] * 6,
        out_specs=[pl.BlockSpec((BN, F), lambda i: (i, 0))] * 2,
        out_shape=[jax.ShapeDtypeStruct((N, F), f32)] * 2,
    )(s, ps[0], ps[1], v, pv[0], pv[1])
    return (out_s, out_v)


# speedup vs baseline: 3.1198x; 1.5795x over previous
import functools

import jax
import jax.numpy as jnp
from jax import lax
from jax.experimental import pallas as pl
from jax.experimental.pallas import tpu as pltpu
from jax.experimental.pallas import tpu_sc as plsc

N = 10000
E = 320000
F = 128
NC = 2
NS = 16
NW = NC * NS
EPW = E // NW
C = 40
NCHUNK = EPW // C
S = 400
SCH = S // C
NSCH = EPW // S
ZCH = C
NZCH = N // ZCH
ZPT = (NZCH + NS - 1) // NS
KG = F // 16


def _phi_kernel(s_ref, w1_ref, b1_ref, w2_ref, b2_ref, os_ref, ovv_ref, ovs_ref):
    h = jnp.dot(s_ref[...], w1_ref[...], preferred_element_type=jnp.float32) + b1_ref[...]
    h = h * jax.nn.sigmoid(h)
    ph = jnp.dot(h, w2_ref[...], preferred_element_type=jnp.float32) + b2_ref[...]
    os_ref[...] = ph[:, :F]
    ovv_ref[...] = ph[:, F:2 * F]
    ovs_ref[...] = ph[:, 2 * F:]


def _wm_kernel(rad_ref, fc_ref, fu_ref, wr_ref, br_ref, ws_ref, wvv_ref, wvs_ref):
    w = jnp.dot(rad_ref[...], wr_ref[...], preferred_element_type=jnp.float32) + br_ref[...]
    fc = jnp.transpose(fc_ref[0], (1, 0))
    fu = jnp.transpose(fu_ref[0], (1, 0))
    ws_ref[...] = w[:, :F] * fc
    wvv_ref[...] = w[:, F:2 * F] * fc
    wvs_ref[...] = w[:, 2 * F:] * (fc * fu)


def _comb_kernel(s_ref, a0_ref, a1_ref, v_ref, c0_ref, c1_ref, os_ref, ov_ref):
    os_ref[...] = s_ref[...] + a0_ref[...] + a1_ref[...]
    ov_ref[...] = v_ref[...] + c0_ref[...] + c1_ref[...]


def _sc_body(phis, phivv, phivs, vtab, ws, wvv, wvs, idxi_hbm, idxj_hbm,
             outs, outv,
             idxi_sup, idxj_sup, idxi_buf, p0, p1, q0, q1, w0, w1, xb, acc,
             semg0, semg1, semw0, semw1):
    ci = lax.axis_index("c")
    si = lax.axis_index("s")
    wid = si * NC + ci
    wbase = wid * EPW

    zeros16 = jnp.zeros((16,), jnp.float32)

    def zero_acc():
        def zfill(j, _):
            for k in range(KG):
                xb[j, pl.ds(k * 16, 16)] = zeros16
            return 0

        lax.fori_loop(0, ZCH, zfill, 0)
        for t in range(ZPT):
            q = si * ZPT + t
            @pl.when(q < NZCH)
            def _():
                pltpu.sync_copy(xb, acc.at[pl.ds(q * ZCH, ZCH)])

    def drain(dst):
        for t in range(ZPT):
            q = si * ZPT + t
            @pl.when(q < NZCH)
            def _():
                pltpu.sync_copy(acc.at[pl.ds(q * ZCH, ZCH)], xb)
                pltpu.sync_copy(xb, dst.at[ci, pl.ds(q * ZCH, ZCH)])

    sets = ((p0, q0, w0, semg0, semw0), (p1, q1, w1, semg1, semw1))

    def phase(g1tab, g2tab, wtab):
        def fire(sbase, k, st):
            p, q, w, semg, semw = st
            off = k * C
            pltpu.async_copy(g1tab.at[idxj_sup.at[pl.ds(off, C)]], p, semg)
            if g2tab is not None:
                pltpu.async_copy(g2tab.at[idxj_sup.at[pl.ds(off, C)]], q, semg)
            pltpu.async_copy(wtab.at[pl.ds(sbase + off, C)], w, semw)

        def wait(st):
            p, q, w, semg, semw = st
            pltpu.make_async_copy(g1tab.at[idxj_sup.at[pl.ds(0, C)]], p, semg).wait()
            if g2tab is not None:
                pltpu.make_async_copy(g2tab.at[idxj_sup.at[pl.ds(0, C)]], q, semg).wait()
            pltpu.make_async_copy(wtab.at[pl.ds(0, C)], w, semw).wait()

        def compute_scatter(k, st):
            p, q, w, semg, semw = st

            def mulrow(j, _):
                for kk in range(KG):
                    sl = pl.ds(kk * 16, 16)
                    if g2tab is not None:
                        xb[j, sl] = q[j, sl] * (p[j, sl] * w[j, sl])
                    else:
                        xb[j, sl] = p[j, sl] * w[j, sl]
                return 0

            lax.fori_loop(0, C, mulrow, 0)
            off = k * C
            for (src_o, dst_o) in ((0, 0), (16, 16), (C - 16, C - 16)):
                idxi_buf[pl.ds(dst_o, 16)] = idxi_sup[pl.ds(off + src_o, 16)]
            pltpu.sync_copy(xb, acc.at[idxi_buf], add=True)

        def super_body(sc, _):
            sbase = wbase + sc * S
            pltpu.sync_copy(idxj_hbm.at[pl.ds(sbase, S)], idxj_sup)
            pltpu.sync_copy(idxi_hbm.at[pl.ds(sbase, S)], idxi_sup)
            fire(sbase, 0, sets[0])

            def pair(m, _):
                fire(sbase, 2 * m + 1, sets[1])
                wait(sets[0])
                compute_scatter(2 * m, sets[0])

                @pl.when(m < SCH // 2 - 1)
                def _():
                    fire(sbase, 2 * m + 2, sets[0])

                wait(sets[1])
                compute_scatter(2 * m + 1, sets[1])
                return 0

            lax.fori_loop(0, SCH // 2, pair, 0)
            return 0

        lax.fori_loop(0, NSCH, super_body, 0)

    zero_acc()
    plsc.subcore_barrier()
    phase(phis, None, ws)
    plsc.subcore_barrier()
    drain(outs)
    plsc.subcore_barrier()

    zero_acc()
    plsc.subcore_barrier()
    phase(phivv, vtab, wvv)
    phase(phivs, None, wvs)
    plsc.subcore_barrier()
    drain(outv)


def kernel(s, v, radial_embeddings, f_cut, unit_vectors, edge_index, W1, b1, W2, b2, Wr, br):
    f32 = jnp.float32

    BN = 1000
    phi_s, phi_vv, phi_vs = pl.pallas_call(
        _phi_kernel,
        grid=(N // BN,),
        in_specs=[
            pl.BlockSpec((BN, F), lambda i: (i, 0)),
            pl.BlockSpec((F, F), lambda i: (0, 0)),
            pl.BlockSpec((1, F), lambda i: (0, 0)),
            pl.BlockSpec((F, 3 * F), lambda i: (0, 0)),
            pl.BlockSpec((1, 3 * F), lambda i: (0, 0)),
        ],
        out_specs=[pl.BlockSpec((BN, F), lambda i: (i, 0))] * 3,
        out_shape=[jax.ShapeDtypeStruct((N, F), f32)] * 3,
    )(s, W1, b1.reshape(1, F), W2, b2.reshape(1, 3 * F))

    BE = 4000
    R = radial_embeddings.shape[1]
    w_s, w_vv, w_vs = pl.pallas_call(
        _wm_kernel,
        grid=(E // BE,),
        in_specs=[
            pl.BlockSpec((BE, R), lambda i: (i, 0)),
            pl.BlockSpec((1, 1, BE), lambda i: (i, 0, 0)),
            pl.BlockSpec((1, 1, BE), lambda i: (i, 0, 0)),
            pl.BlockSpec((R, 3 * F), lambda i: (0, 0)),
            pl.BlockSpec((1, 3 * F), lambda i: (0, 0)),
        ],
        out_specs=[pl.BlockSpec((BE, F), lambda i: (i, 0))] * 3,
        out_shape=[jax.ShapeDtypeStruct((E, F), f32)] * 3,
    )(radial_embeddings, f_cut.reshape(E // BE, 1, BE), unit_vectors.reshape(E // BE, 1, BE),
      Wr, br.reshape(1, 3 * F))

    idx_i = edge_index[0]
    idx_j = edge_index[1]
    mesh = plsc.VectorSubcoreMesh(core_axis_name="c", subcore_axis_name="s")
    ps, pv = pl.kernel(
        _sc_body,
        out_type=(jax.ShapeDtypeStruct((NC, N, F), f32),
                  jax.ShapeDtypeStruct((NC, N, F), f32)),
        mesh=mesh,
        scratch_types=[
            pltpu.VMEM((S,), jnp.int32),
            pltpu.VMEM((S,), jnp.int32),
            pltpu.VMEM((C,), jnp.int32),
            pltpu.VMEM((C, F), f32),
            pltpu.VMEM((C, F), f32),
            pltpu.VMEM((C, F), f32),
            pltpu.VMEM((C, F), f32),
            pltpu.VMEM((C, F), f32),
            pltpu.VMEM((C, F), f32),
            pltpu.VMEM((C, F), f32),
            pltpu.VMEM_SHARED((N, F), f32),
            pltpu.SemaphoreType.DMA,
            pltpu.SemaphoreType.DMA,
            pltpu.SemaphoreType.DMA,
            pltpu.SemaphoreType.DMA,
        ],
    )(phi_s, phi_vv, phi_vs, v, w_s, w_vv, w_vs, idx_i, idx_j)

    out_s, out_v = pl.pallas_call(
        _comb_kernel,
        grid=(N // BN,),
        in_specs=[pl.BlockSpec((BN, F), lambda i: (i, 0))] * 6,
        out_specs=[pl.BlockSpec((BN, F), lambda i: (i, 0))] * 2,
        out_shape=[jax.ShapeDtypeStruct((N, F), f32)] * 2,
    )(s, ps[0], ps[1], v, pv[0], pv[1])
    return (out_s, out_v)


# speedup vs baseline: 3.8395x; 1.2307x over previous
import functools

import jax
import jax.numpy as jnp
from jax import lax
from jax.experimental import pallas as pl
from jax.experimental.pallas import tpu as pltpu
from jax.experimental.pallas import tpu_sc as plsc

N = 10000
E = 320000
F = 128
NC = 2
NS = 16
NW = NC * NS
EPW = E // NW
C = 40
NCHUNK = EPW // C
S = 400
SCH = S // C
NSCH = EPW // S
ZCH = C
NZCH = N // ZCH
ZPT = (NZCH + NS - 1) // NS
KG = F // 16


def _phi_kernel(s_ref, w1_ref, b1_ref, w2_ref, b2_ref, os_ref, ovv_ref, ovs_ref):
    h = jnp.dot(s_ref[...], w1_ref[...], preferred_element_type=jnp.float32) + b1_ref[...]
    h = h * jax.nn.sigmoid(h)
    ph = jnp.dot(h, w2_ref[...], preferred_element_type=jnp.float32) + b2_ref[...]
    os_ref[...] = ph[:, :F]
    ovv_ref[...] = ph[:, F:2 * F]
    ovs_ref[...] = ph[:, 2 * F:]


def _wm_kernel(rad_ref, fc_ref, fu_ref, wr_ref, br_ref, ws_ref, wvv_ref, wvs_ref):
    w = jax.lax.dot_general(rad_ref[...], wr_ref[...], (((0,), (0,)), ((), ())),
                            preferred_element_type=jnp.float32) + br_ref[...]
    fc = jnp.transpose(fc_ref[0], (1, 0))
    fu = jnp.transpose(fu_ref[0], (1, 0))
    ws_ref[...] = w[:, :F] * fc
    wvv_ref[...] = w[:, F:2 * F] * fc
    wvs_ref[...] = w[:, 2 * F:] * (fc * fu)


def _comb_kernel(s_ref, a0_ref, a1_ref, v_ref, c0_ref, c1_ref, os_ref, ov_ref):
    os_ref[...] = s_ref[...] + a0_ref[...] + a1_ref[...]
    ov_ref[...] = v_ref[...] + c0_ref[...] + c1_ref[...]


def _sc_body(phis, phivv, phivs, vtab, ws, wvv, wvs, idxi_hbm, idxj_hbm,
             outs, outv,
             idxi_sup, idxj_sup, ib0, ib1, p0, p1, q0, q1, w0, w1, xb0, xb1, acc,
             semg0, semg1, semw0, semw1, semx0, semx1):
    ci = lax.axis_index("c")
    si = lax.axis_index("s")
    wid = si * NC + ci
    wbase = wid * EPW

    zeros16 = jnp.zeros((16,), jnp.float32)

    def zero_acc():
        def zfill(j, _):
            for k in range(KG):
                xb0[j, pl.ds(k * 16, 16)] = zeros16
            return 0

        lax.fori_loop(0, ZCH, zfill, 0)
        for t in range(ZPT):
            q = si * ZPT + t
            @pl.when(q < NZCH)
            def _():
                pltpu.sync_copy(xb0, acc.at[pl.ds(q * ZCH, ZCH)])

    def drain(dst):
        for t in range(ZPT):
            q = si * ZPT + t
            @pl.when(q < NZCH)
            def _():
                pltpu.sync_copy(acc.at[pl.ds(q * ZCH, ZCH)], xb0)
                pltpu.sync_copy(xb0, dst.at[ci, pl.ds(q * ZCH, ZCH)])

    sets = ((p0, q0, w0, xb0, ib0, semg0, semw0, semx0),
            (p1, q1, w1, xb1, ib1, semg1, semw1, semx1))

    def phase(g1tab, g2tab, wtab):
        def fire(sbase, k, st):
            p, q, w, xb, ib, semg, semw, semx = st
            off = k * C
            pltpu.async_copy(g1tab.at[idxj_sup.at[pl.ds(off, C)]], p, semg)
            if g2tab is not None:
                pltpu.async_copy(g2tab.at[idxj_sup.at[pl.ds(off, C)]], q, semg)
            pltpu.async_copy(wtab.at[pl.ds(sbase + off, C)], w, semw)

        def wait(st):
            p, q, w, xb, ib, semg, semw, semx = st
            pltpu.make_async_copy(g1tab.at[idxj_sup.at[pl.ds(0, C)]], p, semg).wait()
            if g2tab is not None:
                pltpu.make_async_copy(g2tab.at[idxj_sup.at[pl.ds(0, C)]], q, semg).wait()
            pltpu.make_async_copy(wtab.at[pl.ds(0, C)], w, semw).wait()

        def wait_scatter(st):
            p, q, w, xb, ib, semg, semw, semx = st
            pltpu.make_async_copy(xb, acc.at[ib], semx).wait()

        def compute_scatter(k, st, first):
            p, q, w, xb, ib, semg, semw, semx = st

            @pl.when(jnp.logical_not(first))
            def _():
                wait_scatter(st)

            @functools.partial(plsc.parallel_loop, 0, C, unroll=4)
            def mulrow(j):
                for kk in range(KG):
                    sl = pl.ds(kk * 16, 16)
                    if g2tab is not None:
                        xb[j, sl] = q[j, sl] * (p[j, sl] * w[j, sl])
                    else:
                        xb[j, sl] = p[j, sl] * w[j, sl]

            off = k * C
            for (src_o, dst_o) in ((0, 0), (16, 16), (C - 16, C - 16)):
                ib[pl.ds(dst_o, 16)] = idxi_sup[pl.ds(off + src_o, 16)]
            pltpu.async_copy(xb, acc.at[ib], semx, add=True)

        def super_body(sc, _):
            sbase = wbase + sc * S
            pltpu.sync_copy(idxj_hbm.at[pl.ds(sbase, S)], idxj_sup)
            pltpu.sync_copy(idxi_hbm.at[pl.ds(sbase, S)], idxi_sup)
            fire(sbase, 0, sets[0])

            def pair(m, _):
                first = jnp.logical_and(sc == 0, m == 0)
                fire(sbase, 2 * m + 1, sets[1])
                wait(sets[0])
                compute_scatter(2 * m, sets[0], first)

                @pl.when(m < SCH // 2 - 1)
                def _():
                    fire(sbase, 2 * m + 2, sets[0])

                wait(sets[1])
                compute_scatter(2 * m + 1, sets[1], first)
                return 0

            lax.fori_loop(0, SCH // 2, pair, 0)
            return 0

        lax.fori_loop(0, NSCH, super_body, 0)
        wait_scatter(sets[0])
        wait_scatter(sets[1])

    zero_acc()
    plsc.subcore_barrier()
    phase(phis, None, ws)
    plsc.subcore_barrier()
    drain(outs)
    plsc.subcore_barrier()

    zero_acc()
    plsc.subcore_barrier()
    phase(phivv, vtab, wvv)
    phase(phivs, None, wvs)
    plsc.subcore_barrier()
    drain(outv)


def kernel(s, v, radial_embeddings, f_cut, unit_vectors, edge_index, W1, b1, W2, b2, Wr, br):
    f32 = jnp.float32

    BN = 1000
    phi_s, phi_vv, phi_vs = pl.pallas_call(
        _phi_kernel,
        grid=(N // BN,),
        in_specs=[
            pl.BlockSpec((BN, F), lambda i: (i, 0)),
            pl.BlockSpec((F, F), lambda i: (0, 0)),
            pl.BlockSpec((1, F), lambda i: (0, 0)),
            pl.BlockSpec((F, 3 * F), lambda i: (0, 0)),
            pl.BlockSpec((1, 3 * F), lambda i: (0, 0)),
        ],
        out_specs=[pl.BlockSpec((BN, F), lambda i: (i, 0))] * 3,
        out_shape=[jax.ShapeDtypeStruct((N, F), f32)] * 3,
    )(s, W1, b1.reshape(1, F), W2, b2.reshape(1, 3 * F))

    BE = 2560
    R = radial_embeddings.shape[1]
    w_s, w_vv, w_vs = pl.pallas_call(
        _wm_kernel,
        grid=(E // BE,),
        in_specs=[
            pl.BlockSpec((R, BE), lambda i: (0, i)),
            pl.BlockSpec((1, 1, BE), lambda i: (i, 0, 0)),
            pl.BlockSpec((1, 1, BE), lambda i: (i, 0, 0)),
            pl.BlockSpec((R, 3 * F), lambda i: (0, 0)),
            pl.BlockSpec((1, 3 * F), lambda i: (0, 0)),
        ],
        out_specs=[pl.BlockSpec((BE, F), lambda i: (i, 0))] * 3,
        out_shape=[jax.ShapeDtypeStruct((E, F), f32)] * 3,
    )(radial_embeddings.T, f_cut.reshape(E // BE, 1, BE), unit_vectors.reshape(E // BE, 1, BE),
      Wr, br.reshape(1, 3 * F))

    idx_i = edge_index[0]
    idx_j = edge_index[1]
    mesh = plsc.VectorSubcoreMesh(core_axis_name="c", subcore_axis_name="s")
    ps, pv = pl.kernel(
        _sc_body,
        out_type=(jax.ShapeDtypeStruct((NC, N, F), f32),
                  jax.ShapeDtypeStruct((NC, N, F), f32)),
        mesh=mesh,
        scratch_types=[
            pltpu.VMEM((S,), jnp.int32),
            pltpu.VMEM((S,), jnp.int32),
            pltpu.VMEM((C,), jnp.int32),
            pltpu.VMEM((C,), jnp.int32),
            pltpu.VMEM((C, F), f32),
            pltpu.VMEM((C, F), f32),
            pltpu.VMEM((C, F), f32),
            pltpu.VMEM((C, F), f32),
            pltpu.VMEM((C, F), f32),
            pltpu.VMEM((C, F), f32),
            pltpu.VMEM((C, F), f32),
            pltpu.VMEM((C, F), f32),
            pltpu.VMEM_SHARED((N, F), f32),
            pltpu.SemaphoreType.DMA,
            pltpu.SemaphoreType.DMA,
            pltpu.SemaphoreType.DMA,
            pltpu.SemaphoreType.DMA,
            pltpu.SemaphoreType.DMA,
            pltpu.SemaphoreType.DMA,
        ],
    )(phi_s, phi_vv, phi_vs, v, w_s, w_vv, w_vs, idx_i, idx_j)

    out_s, out_v = pl.pallas_call(
        _comb_kernel,
        grid=(N // BN,),
        in_specs=[pl.BlockSpec((BN, F), lambda i: (i, 0))] * 6,
        out_specs=[pl.BlockSpec((BN, F), lambda i: (i, 0))] * 2,
        out_shape=[jax.ShapeDtypeStruct((N, F), f32)] * 2,
    )(s, ps[0], ps[1], v, pv[0], pv[1])
    return (out_s, out_v)
